# baseline (speedup 1.0000x reference)
import functools

import jax
import jax.numpy as jnp
from jax import lax
from jax.experimental import pallas as pl
from jax.experimental.pallas import tpu as pltpu
from jax.experimental.pallas import tpu_sc as plsc

N = 10000
E = 320000
D = 128
H = 128

NC = 2
NS = 16
NW = NC * NS
HD = D // NC
B = 100
NCH1 = E // NS // B
NCH2 = E // NW // B
CW = 16
NP = 10240
RPT = NP // NS
ZR = 80


def _zero_rows(ref, nrows, ncols):
    z16 = jnp.zeros((16,), jnp.float32)

    @pl.loop(0, nrows, step=5)
    def _(i):
        for r in range(5):
            for j in range(0, ncols, 16):
                ref[i + r, pl.ds(j, 16)] = z16


_NB = 4
_GL = 2


def _run_pipeline(nch, start_g, wait_g, start_s, wait_s, ones_fn):
    pro = _NB + (nch % _NB)

    def visit(j, sl, sln, jm2, do_ow, do_g, do_sw):
        wait_g(sl, j)
        start_s(sl, j)
        if ones_fn is not None:
            ones_fn(j, jm2, do_ow)
        if do_g:
            if do_sw:
                wait_s(sln)
            start_g(sln, j + _GL)

    for j in range(_GL):
        start_g(j % _NB, j)
    for j in range(pro):
        visit(j, j % _NB, (j + _GL) % _NB, j % 2,
              j >= 2, j + _GL < nch, j >= _GL)

    @pl.loop(pro, nch - _NB, step=_NB)
    def _(t):
        for k in range(_NB):
            visit(t + k, (pro + k) % _NB, (pro + k + _GL) % _NB,
                  (pro + k) % 2, True, True, True)

    for j in range(nch - _NB, nch):
        visit(j, j % _NB, (j + _GL) % _NB, j % 2,
              True, j + _GL < nch, True)

    for sl in range(_NB):
        wait_s(sl)


def _make_sc_pass1():
    mesh = plsc.VectorSubcoreMesh(core_axis_name="c", subcore_axis_name="s")
    out_type = (jax.ShapeDtypeStruct((NC, NP, HD), jnp.float32),
                jax.ShapeDtypeStruct((NP, CW), jnp.float32))
    scratch = (
        [pltpu.VMEM((NCH1, B), jnp.int32),
         pltpu.VMEM((NCH1, B), jnp.int32),
         pltpu.VMEM((_NB, B, HD), jnp.float32),
         pltpu.VMEM((ZR, HD), jnp.float32),
         pltpu.VMEM((ZR, CW), jnp.float32),
         pltpu.VMEM((B, CW), jnp.float32),
         pltpu.VMEM_SHARED((NP, HD), jnp.float32),
         pltpu.VMEM_SHARED((NP, CW), jnp.float32)]
        + [pltpu.SemaphoreType.DMA] * (2 * _NB + 2))

    @functools.partial(pl.kernel, out_type=out_type, mesh=mesh,
                       scratch_types=scratch,
                       compiler_params=pltpu.CompilerParams(
                           use_tc_tiling_on_sc=False))
    def sc_pass1(table, srcs_hbm, dst_hbm, acc_hbm, cnt_hbm,
                 srcv, dstv, rows, zb, zbc, ones, acc_sh, cnt_sh, *sems):
        gsems = sems[:_NB]
        ssems = sems[_NB:2 * _NB]
        osems = sems[2 * _NB:]
        c = lax.axis_index("c")
        s = lax.axis_index("s")

        _zero_rows(zb, ZR, HD)
        for k in range(RPT // ZR):
            pltpu.sync_copy(zb, acc_sh.at[pl.ds(s * RPT + k * ZR, ZR)])

        @pl.when(c == 0)
        def _():
            _zero_rows(zbc, ZR, CW)
            for k in range(RPT // ZR):
                pltpu.sync_copy(zbc, cnt_sh.at[pl.ds(s * RPT + k * ZR, ZR)])
            o16 = jnp.ones((16,), jnp.float32)

            @pl.loop(0, B, step=5)
            def _(i):
                for r in range(5):
                    ones[i + r, pl.ds(0, CW)] = o16

        pltpu.sync_copy(srcs_hbm.at[c, s], srcv)
        pltpu.sync_copy(dst_hbm.at[s], dstv)
        plsc.subcore_barrier()

        def start_g(sl, j):
            pltpu.async_copy(table.at[srcv.at[j]], rows.at[sl], gsems[sl])

        def wait_g(sl, j):
            pltpu.make_async_copy(
                table.at[srcv.at[j]], rows.at[sl], gsems[sl]).wait()

        def start_s(sl, j):
            pltpu.async_copy(rows.at[sl], acc_sh.at[dstv.at[j]], ssems[sl],
                             add=True)

        def wait_s(sl):
            pltpu.make_async_copy(
                rows.at[sl], acc_sh.at[dstv.at[0]], ssems[sl]).wait()

        def ones_fn(j, jm2, do_ow):
            @pl.when(c == 0)
            def _():
                if do_ow:
                    pltpu.make_async_copy(
                        ones, cnt_sh.at[dstv.at[0]], osems[jm2]).wait()
                pltpu.async_copy(ones, cnt_sh.at[dstv.at[j]], osems[jm2],
                                 add=True)

        _run_pipeline(NCH1, start_g, wait_g, start_s, wait_s, ones_fn)

        @pl.when(c == 0)
        def _():
            for p in range(2):
                pltpu.make_async_copy(
                    ones, cnt_sh.at[dstv.at[0]], osems[p]).wait()

        plsc.subcore_barrier()
        pltpu.sync_copy(acc_sh.at[pl.ds(s * RPT, RPT)],
                        acc_hbm.at[c, pl.ds(s * RPT, RPT)])

        @pl.when(c == 0)
        def _():
            pltpu.sync_copy(cnt_sh.at[pl.ds(s * RPT, RPT)],
                            cnt_hbm.at[pl.ds(s * RPT, RPT)])

    return sc_pass1


def _make_sc_pass2():
    mesh = plsc.VectorSubcoreMesh(core_axis_name="c", subcore_axis_name="s")
    out_type = (jax.ShapeDtypeStruct((NC, NP, CW), jnp.float32),)
    scratch = (
        [pltpu.VMEM((NCH2, B), jnp.int32),
         pltpu.VMEM((NCH2, B), jnp.int32),
         pltpu.VMEM((_NB, B, CW), jnp.float32),
         pltpu.VMEM((ZR, CW), jnp.float32),
         pltpu.VMEM_SHARED((NP, CW), jnp.float32)]
        + [pltpu.SemaphoreType.DMA] * (2 * _NB))

    @functools.partial(pl.kernel, out_type=out_type, mesh=mesh,
                       scratch_types=scratch,
                       compiler_params=pltpu.CompilerParams(
                           use_tc_tiling_on_sc=False))
    def sc_pass2(table, src_hbm, dst_hbm, z_hbm,
                 srcv, dstv, rows, zbc, acc_sh, *sems):
        gsems = sems[:_NB]
        ssems = sems[_NB:]
        c = lax.axis_index("c")
        s = lax.axis_index("s")
        wid = c * NS + s

        _zero_rows(zbc, ZR, CW)
        for k in range(RPT // ZR):
            pltpu.sync_copy(zbc, acc_sh.at[pl.ds(s * RPT + k * ZR, ZR)])

        pltpu.sync_copy(src_hbm.at[wid], srcv)
        pltpu.sync_copy(dst_hbm.at[wid], dstv)
        plsc.subcore_barrier()

        def start_g(sl, j):
            pltpu.async_copy(table.at[srcv.at[j]], rows.at[sl], gsems[sl])

        def wait_g(sl, j):
            pltpu.make_async_copy(
                table.at[srcv.at[j]], rows.at[sl], gsems[sl]).wait()

        def start_s(sl, j):
            pltpu.async_copy(rows.at[sl], acc_sh.at[dstv.at[j]], ssems[sl],
                             add=True)

        def wait_s(sl):
            pltpu.make_async_copy(
                rows.at[sl], acc_sh.at[dstv.at[0]], ssems[sl]).wait()

        _run_pipeline(NCH2, start_g, wait_g, start_s, wait_s, None)

        plsc.subcore_barrier()
        pltpu.sync_copy(acc_sh.at[pl.ds(s * RPT, RPT)],
                        z_hbm.at[c, pl.ds(s * RPT, RPT)])

    return sc_pass2


_sc_aggregate_x = _make_sc_pass1()
_sc_aggregate_y = _make_sc_pass2()

_DN = (((1,), (1,)), ((), ()))
_PREC = lax.Precision.HIGHEST
_RB = 1000


def _kb_body(acc_ref, cnt_ref, x_ref, w1r_ref, b1_ref, w1l_ref, w2l_ref,
             w2r_ref, yb_ref, r2_ref, invc_ref):
    inv = 1.0 / jnp.maximum(cnt_ref[:, 0:1], 1.0)
    aggrn = jnp.concatenate([acc_ref[0], acc_ref[1]], axis=1) * inv
    r1 = (lax.dot_general(x_ref[...], w1r_ref[...], _DN, precision=_PREC)
          + b1_ref[...])
    t = lax.dot_general(aggrn, w1l_ref[...], _DN, precision=_PREC) + r1
    h = jnp.maximum(t, 0.0)
    y = lax.dot_general(h, w2l_ref[...], _DN, precision=_PREC)
    r2 = lax.dot_general(h, w2r_ref[...], _DN, precision=_PREC)
    yb_ref[...] = jnp.broadcast_to(y, (y.shape[0], CW))
    r2_ref[...] = r2
    invc_ref[...] = inv


def _kc_body(z_ref, invc_ref, r2_ref, b2_ref, o_ref):
    z = z_ref[0][:, 0:1] + z_ref[1][:, 0:1]
    o_ref[...] = jax.nn.sigmoid(z * invc_ref[...] + r2_ref[...] + b2_ref[...])


def kernel(x, edge_index, W1_l, b1_l, W1_r, W2_l, b2_l, W2_r):
    src = edge_index[0].astype(jnp.int32)
    dst = edge_index[1].astype(jnp.int32)
    src2 = src * 2
    srcs1 = jnp.stack([src2, src2 + 1]).reshape(NC, NS, NCH1, B)
    dst1 = dst.reshape(NS, NCH1, B)
    src_p2 = src.reshape(NW, NCH2, B)
    dst_p2 = dst.reshape(NW, NCH2, B)
    table1 = x.reshape(N * NC, HD)

    acc, cnt = _sc_aggregate_x(table1, srcs1, dst1)

    yb, r2, invc = pl.pallas_call(
        _kb_body,
        out_shape=(jax.ShapeDtypeStruct((N, CW), jnp.float32),
                   jax.ShapeDtypeStruct((N, 1), jnp.float32),
                   jax.ShapeDtypeStruct((N, 1), jnp.float32)),
        grid=(N // _RB,),
        in_specs=[pl.BlockSpec((NC, _RB, HD), lambda i: (0, i, 0)),
                  pl.BlockSpec((_RB, CW), lambda i: (i, 0)),
                  pl.BlockSpec((_RB, D), lambda i: (i, 0)),
                  pl.BlockSpec((H, D), lambda i: (0, 0)),
                  pl.BlockSpec((1, H), lambda i: (0, 0)),
                  pl.BlockSpec((H, D), lambda i: (0, 0)),
                  pl.BlockSpec((1, H), lambda i: (0, 0)),
                  pl.BlockSpec((1, H), lambda i: (0, 0))],
        out_specs=(pl.BlockSpec((_RB, CW), lambda i: (i, 0)),
                   pl.BlockSpec((_RB, 1), lambda i: (i, 0)),
                   pl.BlockSpec((_RB, 1), lambda i: (i, 0))),
    )(acc, cnt, x, W1_r, b1_l.reshape(1, H), W1_l, W2_l, W2_r)

    (z,) = _sc_aggregate_y(yb, src_p2, dst_p2)

    out = pl.pallas_call(
        _kc_body,
        out_shape=jax.ShapeDtypeStruct((N, 1), jnp.float32),
        grid=(N // _RB,),
        in_specs=[pl.BlockSpec((NC, _RB, CW), lambda i: (0, i, 0)),
                  pl.BlockSpec((_RB, 1), lambda i: (i, 0)),
                  pl.BlockSpec((_RB, 1), lambda i: (i, 0)),
                  pl.BlockSpec((1, 1), lambda i: (0, 0))],
        out_specs=pl.BlockSpec((_RB, 1), lambda i: (i, 0)),
    )(z, invc, r2, b2_l.reshape(1, 1))
    return out


# speedup vs baseline: 1.0128x; 1.0128x over previous
import functools

import jax
import jax.numpy as jnp
from jax import lax
from jax.experimental import pallas as pl
from jax.experimental.pallas import tpu as pltpu
from jax.experimental.pallas import tpu_sc as plsc

N = 10000
E = 320000
D = 128
H = 128

NC = 2
NS = 16
NW = NC * NS
HD = D // NC
B = 100
NCH1 = E // NS // B
NCH2 = E // NW // B
CW = 16
NP = 10240
RPT = NP // NS
ZR = 80


def _zero_rows(ref, nrows, ncols):
    z16 = jnp.zeros((16,), jnp.float32)

    @pl.loop(0, nrows, step=5)
    def _(i):
        for r in range(5):
            for j in range(0, ncols, 16):
                ref[i + r, pl.ds(j, 16)] = z16


_NB = 4
_GL = 2


def _run_pipeline(nch, start_g, wait_g, start_s, wait_s, ones_fn):
    pro = _NB + (nch % _NB)

    def visit(j, sl, sln, jm2, do_ow, do_g, do_sw):
        wait_g(sl, j)
        start_s(sl, j)
        if ones_fn is not None:
            ones_fn(j, jm2, do_ow)
        if do_g:
            if do_sw:
                wait_s(sln)
            start_g(sln, j + _GL)

    for j in range(_GL):
        start_g(j % _NB, j)
    for j in range(pro):
        visit(j, j % _NB, (j + _GL) % _NB, j % 2,
              j >= 2, j + _GL < nch, j >= _GL)

    @pl.loop(pro, nch - _NB, step=_NB)
    def _(t):
        for k in range(_NB):
            visit(t + k, (pro + k) % _NB, (pro + k + _GL) % _NB,
                  (pro + k) % 2, True, True, True)

    for j in range(nch - _NB, nch):
        visit(j, j % _NB, (j + _GL) % _NB, j % 2,
              True, j + _GL < nch, True)

    for sl in range(_NB):
        wait_s(sl)


def _make_sc_pass1():
    mesh = plsc.VectorSubcoreMesh(core_axis_name="c", subcore_axis_name="s")
    out_type = (jax.ShapeDtypeStruct((NC, NP, HD), jnp.float32),
                jax.ShapeDtypeStruct((NP, CW), jnp.float32))
    scratch = (
        [pltpu.VMEM((NCH1, B), jnp.int32),
         pltpu.VMEM((NCH1, B), jnp.int32),
         pltpu.VMEM((_NB, B, HD), jnp.float32),
         pltpu.VMEM((ZR, HD), jnp.float32),
         pltpu.VMEM((ZR, CW), jnp.float32),
         pltpu.VMEM((B, CW), jnp.float32),
         pltpu.VMEM_SHARED((NP, HD), jnp.float32),
         pltpu.VMEM_SHARED((NP, CW), jnp.float32)]
        + [pltpu.SemaphoreType.DMA] * (2 * _NB + 2))

    @functools.partial(pl.kernel, out_type=out_type, mesh=mesh,
                       scratch_types=scratch,
                       compiler_params=pltpu.CompilerParams(
                           use_tc_tiling_on_sc=False))
    def sc_pass1(table, srcs_hbm, dst_hbm, acc_hbm, cnt_hbm,
                 srcv, dstv, rows, zb, zbc, ones, acc_sh, cnt_sh, *sems):
        gsems = sems[:_NB]
        ssems = sems[_NB:2 * _NB]
        osems = sems[2 * _NB:]
        c = lax.axis_index("c")
        s = lax.axis_index("s")

        _zero_rows(zb, ZR, HD)
        for k in range(RPT // ZR):
            pltpu.sync_copy(zb, acc_sh.at[pl.ds(s * RPT + k * ZR, ZR)])

        @pl.when(c == 0)
        def _():
            _zero_rows(zbc, ZR, CW)
            for k in range(RPT // ZR):
                pltpu.sync_copy(zbc, cnt_sh.at[pl.ds(s * RPT + k * ZR, ZR)])
            o16 = jnp.ones((16,), jnp.float32)

            @pl.loop(0, B, step=5)
            def _(i):
                for r in range(5):
                    ones[i + r, pl.ds(0, CW)] = o16

        pltpu.sync_copy(srcs_hbm.at[c, s], srcv)
        pltpu.sync_copy(dst_hbm.at[s], dstv)
        plsc.subcore_barrier()

        def start_g(sl, j):
            pltpu.async_copy(table.at[srcv.at[j]], rows.at[sl], gsems[sl])

        def wait_g(sl, j):
            pltpu.make_async_copy(
                table.at[srcv.at[j]], rows.at[sl], gsems[sl]).wait()

        def start_s(sl, j):
            pltpu.async_copy(rows.at[sl], acc_sh.at[dstv.at[j]], ssems[sl],
                             add=True)

        def wait_s(sl):
            pltpu.make_async_copy(
                rows.at[sl], acc_sh.at[dstv.at[0]], ssems[sl]).wait()

        def ones_fn(j, jm2, do_ow):
            @pl.when(c == 0)
            def _():
                if do_ow:
                    pltpu.make_async_copy(
                        ones, cnt_sh.at[dstv.at[0]], osems[jm2]).wait()
                pltpu.async_copy(ones, cnt_sh.at[dstv.at[j]], osems[jm2],
                                 add=True)

        _run_pipeline(NCH1, start_g, wait_g, start_s, wait_s, ones_fn)

        @pl.when(c == 0)
        def _():
            for p in range(2):
                pltpu.make_async_copy(
                    ones, cnt_sh.at[dstv.at[0]], osems[p]).wait()

        plsc.subcore_barrier()
        pltpu.sync_copy(acc_sh.at[pl.ds(s * RPT, RPT)],
                        acc_hbm.at[c, pl.ds(s * RPT, RPT)])

        @pl.when(c == 0)
        def _():
            pltpu.sync_copy(cnt_sh.at[pl.ds(s * RPT, RPT)],
                            cnt_hbm.at[pl.ds(s * RPT, RPT)])

    return sc_pass1


def _make_sc_pass2():
    mesh = plsc.VectorSubcoreMesh(core_axis_name="c", subcore_axis_name="s")
    out_type = (jax.ShapeDtypeStruct((NC, NP, CW), jnp.float32),)
    scratch = (
        [pltpu.VMEM((NCH2, B), jnp.int32),
         pltpu.VMEM((NCH2, B), jnp.int32),
         pltpu.VMEM((_NB, B, CW), jnp.float32),
         pltpu.VMEM((ZR, CW), jnp.float32),
         pltpu.VMEM_SHARED((NP, CW), jnp.float32)]
        + [pltpu.SemaphoreType.DMA] * (2 * _NB))

    @functools.partial(pl.kernel, out_type=out_type, mesh=mesh,
                       scratch_types=scratch,
                       compiler_params=pltpu.CompilerParams(
                           use_tc_tiling_on_sc=False))
    def sc_pass2(table, src_hbm, dst_hbm, z_hbm,
                 srcv, dstv, rows, zbc, acc_sh, *sems):
        gsems = sems[:_NB]
        ssems = sems[_NB:]
        c = lax.axis_index("c")
        s = lax.axis_index("s")
        wid = c * NS + s

        _zero_rows(zbc, ZR, CW)
        for k in range(RPT // ZR):
            pltpu.sync_copy(zbc, acc_sh.at[pl.ds(s * RPT + k * ZR, ZR)])

        pltpu.sync_copy(src_hbm.at[wid], srcv)
        pltpu.sync_copy(dst_hbm.at[wid], dstv)
        plsc.subcore_barrier()

        def start_g(sl, j):
            pltpu.async_copy(table.at[srcv.at[j]], rows.at[sl], gsems[sl])

        def wait_g(sl, j):
            pltpu.make_async_copy(
                table.at[srcv.at[j]], rows.at[sl], gsems[sl]).wait()

        def start_s(sl, j):
            pltpu.async_copy(rows.at[sl], acc_sh.at[dstv.at[j]], ssems[sl],
                             add=True)

        def wait_s(sl):
            pltpu.make_async_copy(
                rows.at[sl], acc_sh.at[dstv.at[0]], ssems[sl]).wait()

        _run_pipeline(NCH2, start_g, wait_g, start_s, wait_s, None)

        plsc.subcore_barrier()
        pltpu.sync_copy(acc_sh.at[pl.ds(s * RPT, RPT)],
                        z_hbm.at[c, pl.ds(s * RPT, RPT)])

    return sc_pass2


_sc_aggregate_x = _make_sc_pass1()
_sc_aggregate_y = _make_sc_pass2()

_DN = (((1,), (1,)), ((), ()))
_PREC = lax.Precision.HIGHEST
_RB = 1000


def _ka_body(x_ref, w_ref, b_ref, o_ref):
    o_ref[...] = (lax.dot_general(x_ref[...], w_ref[...], _DN,
                                  precision=_PREC) + b_ref[...])


def _kb_body(acc_ref, cnt_ref, r1_ref, w1l_ref, w2l_ref, w2r_ref,
             yb_ref, r2_ref, invc_ref):
    inv = 1.0 / jnp.maximum(cnt_ref[:, 0:1], 1.0)
    aggrn = jnp.concatenate([acc_ref[0], acc_ref[1]], axis=1) * inv
    t = lax.dot_general(aggrn, w1l_ref[...], _DN, precision=_PREC) + r1_ref[...]
    h = jnp.maximum(t, 0.0)
    y = lax.dot_general(h, w2l_ref[...], _DN, precision=_PREC)
    r2 = lax.dot_general(h, w2r_ref[...], _DN, precision=_PREC)
    yb_ref[...] = jnp.broadcast_to(y, (y.shape[0], CW))
    r2_ref[...] = r2
    invc_ref[...] = inv


def _kc_body(z_ref, invc_ref, r2_ref, b2_ref, o_ref):
    z = z_ref[0][:, 0:1] + z_ref[1][:, 0:1]
    o_ref[...] = jax.nn.sigmoid(z * invc_ref[...] + r2_ref[...] + b2_ref[...])


def kernel(x, edge_index, W1_l, b1_l, W1_r, W2_l, b2_l, W2_r):
    src = edge_index[0].astype(jnp.int32)
    dst = edge_index[1].astype(jnp.int32)
    src2 = src * 2
    srcs1 = jnp.stack([src2, src2 + 1]).reshape(NC, NS, NCH1, B)
    dst1 = dst.reshape(NS, NCH1, B)
    src_p2 = src.reshape(NW, NCH2, B)
    dst_p2 = dst.reshape(NW, NCH2, B)
    table1 = x.reshape(N * NC, HD)

    r1 = pl.pallas_call(
        _ka_body,
        out_shape=jax.ShapeDtypeStruct((N, H), jnp.float32),
        grid=(N // _RB,),
        in_specs=[pl.BlockSpec((_RB, D), lambda i: (i, 0)),
                  pl.BlockSpec((H, D), lambda i: (0, 0)),
                  pl.BlockSpec((1, H), lambda i: (0, 0))],
        out_specs=pl.BlockSpec((_RB, H), lambda i: (i, 0)),
    )(x, W1_r, b1_l.reshape(1, H))

    acc, cnt = _sc_aggregate_x(table1, srcs1, dst1)

    yb, r2, invc = pl.pallas_call(
        _kb_body,
        out_shape=(jax.ShapeDtypeStruct((N, CW), jnp.float32),
                   jax.ShapeDtypeStruct((N, 1), jnp.float32),
                   jax.ShapeDtypeStruct((N, 1), jnp.float32)),
        grid=(N // _RB,),
        in_specs=[pl.BlockSpec((NC, _RB, HD), lambda i: (0, i, 0)),
                  pl.BlockSpec((_RB, CW), lambda i: (i, 0)),
                  pl.BlockSpec((_RB, H), lambda i: (i, 0)),
                  pl.BlockSpec((H, D), lambda i: (0, 0)),
                  pl.BlockSpec((1, H), lambda i: (0, 0)),
                  pl.BlockSpec((1, H), lambda i: (0, 0))],
        out_specs=(pl.BlockSpec((_RB, CW), lambda i: (i, 0)),
                   pl.BlockSpec((_RB, 1), lambda i: (i, 0)),
                   pl.BlockSpec((_RB, 1), lambda i: (i, 0))),
    )(acc, cnt, r1, W1_l, W2_l, W2_r)

    (z,) = _sc_aggregate_y(yb, src_p2, dst_p2)

    out = pl.pallas_call(
        _kc_body,
        out_shape=jax.ShapeDtypeStruct((N, 1), jnp.float32),
        grid=(N // _RB,),
        in_specs=[pl.BlockSpec((NC, _RB, CW), lambda i: (0, i, 0)),
                  pl.BlockSpec((_RB, 1), lambda i: (i, 0)),
                  pl.BlockSpec((_RB, 1), lambda i: (i, 0)),
                  pl.BlockSpec((1, 1), lambda i: (0, 0))],
        out_specs=pl.BlockSpec((_RB, 1), lambda i: (i, 0)),
    )(z, invc, r2, b2_l.reshape(1, 1))
    return out
